# Initial kernel scaffold; baseline (speedup 1.0000x reference)
import jax
import jax.numpy as jnp
from jax.experimental import pallas as pl


def kernel(x, edge_index, W1, b1, W2, b2):
    raise NotImplementedError("write your pallas kernel here")


# speedup vs baseline: 13.0849x; 13.0849x over previous
import functools

import jax
import jax.numpy as jnp
from jax import lax
from jax.experimental import pallas as pl
from jax.experimental.pallas import tpu as pltpu
from jax.experimental.pallas import tpu_sc as plsc

N = 10000
E = 320000
D = 128
H = 16
C = 7
CP = 16

NC, NS = 2, 16
NW = NC * NS
CH = 125
EPW = E // NW
NCH = EPW // CH
NP = 10240
RPT = NP // NS


def _seg_body(src_hbm, dst_hbm, feats_hbm, zeros_hbm, out_hbm,
              src_v, dst_v, rows_v, acc_sh, sem):
    cid = lax.axis_index("c")
    sid = lax.axis_index("s")
    wid = sid * NC + cid
    pltpu.sync_copy(zeros_hbm, acc_sh.at[pl.ds(sid * RPT, RPT)])
    base = wid * NCH
    pltpu.sync_copy(src_hbm.at[pl.ds(base, NCH)], src_v)
    pltpu.sync_copy(dst_hbm.at[pl.ds(base, NCH)], dst_v)
    plsc.subcore_barrier()

    def step(j, carry):
        pltpu.async_copy(feats_hbm.at[src_v.at[j]], rows_v, sem).wait()
        pltpu.sync_copy(rows_v, acc_sh.at[dst_v.at[j]], add=True)
        return carry

    lax.fori_loop(0, NCH, step, 0)
    plsc.subcore_barrier()
    pltpu.sync_copy(acc_sh.at[pl.ds(sid * RPT, RPT)],
                    out_hbm.at[cid, pl.ds(sid * RPT, RPT)])


def _make_seg(feat):
    mesh = plsc.VectorSubcoreMesh(core_axis_name="c", subcore_axis_name="s",
                                  num_cores=NC, num_subcores=NS)
    return pl.kernel(
        _seg_body,
        out_type=jax.ShapeDtypeStruct((NC, NP, feat), jnp.float32),
        mesh=mesh,
        compiler_params=pltpu.CompilerParams(use_tc_tiling_on_sc=False),
        scratch_types=[
            pltpu.VMEM((NCH, CH), jnp.int32),
            pltpu.VMEM((NCH, CH), jnp.int32),
            pltpu.VMEM((CH, feat), jnp.float32),
            pltpu.VMEM_SHARED((NP, feat), jnp.float32),
            pltpu.SemaphoreType.DMA,
        ],
    )


_seg16 = _make_seg(H)
_segcp = _make_seg(CP)


def _mm1_body(x_ref, w_ref, o_ref):
    o_ref[...] = jnp.dot(x_ref[...], w_ref[...],
                         preferred_element_type=jnp.float32)


_mm1 = pl.pallas_call(
    _mm1_body,
    grid=(10,),
    in_specs=[pl.BlockSpec((N // 10, D), lambda i: (i, 0)),
              pl.BlockSpec((D, H), lambda i: (0, 0))],
    out_specs=pl.BlockSpec((N // 10, H), lambda i: (i, 0)),
    out_shape=jax.ShapeDtypeStruct((N, H), jnp.float32),
)


def _mid_body(p_ref, b_ref, w_ref, o_ref):
    h = jnp.maximum(p_ref[0] + p_ref[1] + b_ref[...], 0.0)
    o_ref[...] = jnp.dot(h, w_ref[...], preferred_element_type=jnp.float32)


_mid = pl.pallas_call(
    _mid_body,
    grid=(10,),
    in_specs=[pl.BlockSpec((NC, NP // 10, H), lambda i: (0, i, 0)),
              pl.BlockSpec((1, H), lambda i: (0, 0)),
              pl.BlockSpec((H, CP), lambda i: (0, 0))],
    out_specs=pl.BlockSpec((NP // 10, CP), lambda i: (i, 0)),
    out_shape=jax.ShapeDtypeStruct((NP, CP), jnp.float32),
)


def _post_body(q_ref, b_ref, o_ref):
    o_ref[...] = q_ref[0] + q_ref[1] + b_ref[...]


_post = pl.pallas_call(
    _post_body,
    grid=(10,),
    in_specs=[pl.BlockSpec((NC, NP // 10, CP), lambda i: (0, i, 0)),
              pl.BlockSpec((1, CP), lambda i: (0, 0))],
    out_specs=pl.BlockSpec((NP // 10, CP), lambda i: (i, 0)),
    out_shape=jax.ShapeDtypeStruct((NP, CP), jnp.float32),
)


def kernel(x, edge_index, W1, b1, W2, b2):
    src2d = edge_index[0].reshape(E // CH, CH)
    dst2d = edge_index[1].reshape(E // CH, CH)
    z16 = jnp.zeros((RPT, H), jnp.float32)
    zcp = jnp.zeros((RPT, CP), jnp.float32)
    w2p = jnp.pad(W2, ((0, 0), (0, CP - C)))
    b2p = jnp.pad(b2, (0, CP - C)).reshape(1, CP)

    h1 = _mm1(x, W1)
    p1 = _seg16(src2d, dst2d, h1, z16)
    h2 = _mid(p1, b1.reshape(1, H), w2p)
    p2 = _segcp(src2d, dst2d, h2, zcp)
    out = _post(p2, b2p)
    return out[:N, :C]


# speedup vs baseline: 14.2654x; 1.0902x over previous
import functools

import jax
import jax.numpy as jnp
from jax import lax
from jax.experimental import pallas as pl
from jax.experimental.pallas import tpu as pltpu
from jax.experimental.pallas import tpu_sc as plsc

N = 10000
E = 320000
D = 128
H = 16
C = 7
CP = 16

NC, NS = 2, 16
NW = NC * NS
CH = 125
EPW = E // NW
NCH = EPW // CH
NP = 10240
RPT = NP // NS


def _seg_body(src_hbm, dst_hbm, feats_hbm, zeros_hbm, out_hbm,
              src_v, dst_v, rows0, rows1, acc_sh, sem0, sem1):
    cid = lax.axis_index("c")
    sid = lax.axis_index("s")
    wid = sid * NC + cid
    pltpu.sync_copy(zeros_hbm, acc_sh.at[pl.ds(sid * RPT, RPT)])
    base = wid * NCH
    pltpu.sync_copy(src_hbm.at[pl.ds(base, NCH)], src_v)
    pltpu.sync_copy(dst_hbm.at[pl.ds(base, NCH)], dst_v)
    plsc.subcore_barrier()

    pltpu.async_copy(feats_hbm.at[src_v.at[0]], rows0, sem0)
    npairs = NCH // 2

    def step(i, carry):
        j0 = 2 * i
        pltpu.make_async_copy(feats_hbm.at[pl.ds(0, CH)], rows0, sem0).wait()
        pltpu.async_copy(feats_hbm.at[src_v.at[j0 + 1]], rows1, sem1)
        pltpu.sync_copy(rows0, acc_sh.at[dst_v.at[j0]], add=True)
        pltpu.make_async_copy(feats_hbm.at[pl.ds(0, CH)], rows1, sem1).wait()

        @pl.when(i + 1 < npairs)
        def _():
            pltpu.async_copy(feats_hbm.at[src_v.at[j0 + 2]], rows0, sem0)

        pltpu.sync_copy(rows1, acc_sh.at[dst_v.at[j0 + 1]], add=True)
        return carry

    lax.fori_loop(0, npairs, step, 0)
    plsc.subcore_barrier()
    pltpu.sync_copy(acc_sh.at[pl.ds(sid * RPT, RPT)],
                    out_hbm.at[cid, pl.ds(sid * RPT, RPT)])


def _make_seg(feat):
    mesh = plsc.VectorSubcoreMesh(core_axis_name="c", subcore_axis_name="s",
                                  num_cores=NC, num_subcores=NS)
    return pl.kernel(
        _seg_body,
        out_type=jax.ShapeDtypeStruct((NC, NP, feat), jnp.float32),
        mesh=mesh,
        compiler_params=pltpu.CompilerParams(use_tc_tiling_on_sc=False),
        scratch_types=[
            pltpu.VMEM((NCH, CH), jnp.int32),
            pltpu.VMEM((NCH, CH), jnp.int32),
            pltpu.VMEM((CH, feat), jnp.float32),
            pltpu.VMEM((CH, feat), jnp.float32),
            pltpu.VMEM_SHARED((NP, feat), jnp.float32),
            pltpu.SemaphoreType.DMA,
            pltpu.SemaphoreType.DMA,
        ],
    )


_seg16 = _make_seg(H)
_segcp = _make_seg(CP)


def _mm1_body(x_ref, w_ref, o_ref):
    o_ref[...] = jnp.dot(x_ref[...], w_ref[...],
                         preferred_element_type=jnp.float32)


_mm1 = pl.pallas_call(
    _mm1_body,
    grid=(10,),
    in_specs=[pl.BlockSpec((N // 10, D), lambda i: (i, 0)),
              pl.BlockSpec((D, H), lambda i: (0, 0))],
    out_specs=pl.BlockSpec((N // 10, H), lambda i: (i, 0)),
    out_shape=jax.ShapeDtypeStruct((N, H), jnp.float32),
)


def _mid_body(p_ref, b_ref, w_ref, o_ref):
    h = jnp.maximum(p_ref[0] + p_ref[1] + b_ref[...], 0.0)
    o_ref[...] = jnp.dot(h, w_ref[...], preferred_element_type=jnp.float32)


_mid = pl.pallas_call(
    _mid_body,
    grid=(10,),
    in_specs=[pl.BlockSpec((NC, NP // 10, H), lambda i: (0, i, 0)),
              pl.BlockSpec((1, H), lambda i: (0, 0)),
              pl.BlockSpec((H, CP), lambda i: (0, 0))],
    out_specs=pl.BlockSpec((NP // 10, CP), lambda i: (i, 0)),
    out_shape=jax.ShapeDtypeStruct((NP, CP), jnp.float32),
)


def _post_body(q_ref, b_ref, o_ref):
    o_ref[...] = q_ref[0] + q_ref[1] + b_ref[...]


_post = pl.pallas_call(
    _post_body,
    grid=(10,),
    in_specs=[pl.BlockSpec((NC, NP // 10, CP), lambda i: (0, i, 0)),
              pl.BlockSpec((1, CP), lambda i: (0, 0))],
    out_specs=pl.BlockSpec((NP // 10, CP), lambda i: (i, 0)),
    out_shape=jax.ShapeDtypeStruct((NP, CP), jnp.float32),
)


def kernel(x, edge_index, W1, b1, W2, b2):
    src2d = edge_index[0].reshape(E // CH, CH)
    dst2d = edge_index[1].reshape(E // CH, CH)
    z16 = jnp.zeros((RPT, H), jnp.float32)
    zcp = jnp.zeros((RPT, CP), jnp.float32)
    w2p = jnp.pad(W2, ((0, 0), (0, CP - C)))
    b2p = jnp.pad(b2, (0, CP - C)).reshape(1, CP)

    h1 = _mm1(x, W1)
    p1 = _seg16(src2d, dst2d, h1, z16)
    h2 = _mid(p1, b1.reshape(1, H), w2p)
    p2 = _segcp(src2d, dst2d, h2, zcp)
    out = _post(p2, b2p)
    return out[:N, :C]


# speedup vs baseline: 22.4678x; 1.5750x over previous
import functools

import jax
import jax.numpy as jnp
from jax import lax
from jax.experimental import pallas as pl
from jax.experimental.pallas import tpu as pltpu
from jax.experimental.pallas import tpu_sc as plsc

N = 10000
E = 320000
D = 128
H = 16
C = 7
CP = 16

NC, NS = 2, 16
NW = NC * NS
CH = 125
EPW = E // NW
NCH = EPW // CH
NP = 10240
RPT = NP // NS


NBUF = 8
HALF = NBUF // 2


def _seg_body(src_hbm, dst_hbm, feats_hbm, zeros_hbm, out_hbm,
              src_v, dst_v, rows_v, acc_sh, gsem, ssem):
    cid = lax.axis_index("c")
    sid = lax.axis_index("s")
    wid = sid * NC + cid
    pltpu.sync_copy(zeros_hbm, acc_sh.at[pl.ds(sid * RPT, RPT)])
    base = wid * NCH
    pltpu.sync_copy(src_hbm.at[pl.ds(base, NCH)], src_v)
    pltpu.sync_copy(dst_hbm.at[pl.ds(base, NCH)], dst_v)
    plsc.subcore_barrier()

    for b in range(NBUF):
        pltpu.async_copy(feats_hbm.at[src_v.at[b]], rows_v.at[b], gsem.at[b])

    def rnd(r, carry):
        jo = r * NBUF
        for b in range(NBUF):
            j = jo + b
            pltpu.make_async_copy(feats_hbm.at[pl.ds(0, CH)],
                                  rows_v.at[b], gsem.at[b]).wait()
            pltpu.async_copy(rows_v.at[b], acc_sh.at[dst_v.at[j]],
                             ssem.at[b], add=True)
            bq = (b + HALF) % NBUF
            jq = j + HALF

            @pl.when(jnp.logical_and(jq >= NBUF, jq < NCH))
            def _():
                pltpu.make_async_copy(feats_hbm.at[pl.ds(0, CH)],
                                      rows_v.at[bq], ssem.at[bq]).wait()
                pltpu.async_copy(feats_hbm.at[src_v.at[jq]],
                                 rows_v.at[bq], gsem.at[bq])

        return carry

    lax.fori_loop(0, NCH // NBUF, rnd, 0)
    for b in range(NBUF):
        pltpu.make_async_copy(feats_hbm.at[pl.ds(0, CH)],
                              rows_v.at[b], ssem.at[b]).wait()
    plsc.subcore_barrier()
    pltpu.sync_copy(acc_sh.at[pl.ds(sid * RPT, RPT)],
                    out_hbm.at[cid, pl.ds(sid * RPT, RPT)])


def _make_seg(feat):
    mesh = plsc.VectorSubcoreMesh(core_axis_name="c", subcore_axis_name="s",
                                  num_cores=NC, num_subcores=NS)
    return pl.kernel(
        _seg_body,
        out_type=jax.ShapeDtypeStruct((NC, NP, feat), jnp.float32),
        mesh=mesh,
        compiler_params=pltpu.CompilerParams(use_tc_tiling_on_sc=False),
        scratch_types=[
            pltpu.VMEM((NCH, CH), jnp.int32),
            pltpu.VMEM((NCH, CH), jnp.int32),
            pltpu.VMEM((NBUF, CH, feat), jnp.float32),
            pltpu.VMEM_SHARED((NP, feat), jnp.float32),
            pltpu.SemaphoreType.DMA((NBUF,)),
            pltpu.SemaphoreType.DMA((NBUF,)),
        ],
    )


_seg16 = _make_seg(H)
_segcp = _make_seg(CP)


def _mm1_body(x_ref, w_ref, o_ref):
    o_ref[...] = jnp.dot(x_ref[...], w_ref[...],
                         preferred_element_type=jnp.float32)


_mm1 = pl.pallas_call(
    _mm1_body,
    grid=(10,),
    in_specs=[pl.BlockSpec((N // 10, D), lambda i: (i, 0)),
              pl.BlockSpec((D, H), lambda i: (0, 0))],
    out_specs=pl.BlockSpec((N // 10, H), lambda i: (i, 0)),
    out_shape=jax.ShapeDtypeStruct((N, H), jnp.float32),
)


def _mid_body(p_ref, b_ref, w_ref, o_ref):
    h = jnp.maximum(p_ref[0] + p_ref[1] + b_ref[...], 0.0)
    o_ref[...] = jnp.dot(h, w_ref[...], preferred_element_type=jnp.float32)


_mid = pl.pallas_call(
    _mid_body,
    grid=(10,),
    in_specs=[pl.BlockSpec((NC, NP // 10, H), lambda i: (0, i, 0)),
              pl.BlockSpec((1, H), lambda i: (0, 0)),
              pl.BlockSpec((H, CP), lambda i: (0, 0))],
    out_specs=pl.BlockSpec((NP // 10, CP), lambda i: (i, 0)),
    out_shape=jax.ShapeDtypeStruct((NP, CP), jnp.float32),
)


def _post_body(q_ref, b_ref, o_ref):
    o_ref[...] = (q_ref[0] + q_ref[1] + b_ref[...])[:, :C]


_post = pl.pallas_call(
    _post_body,
    grid=(10,),
    in_specs=[pl.BlockSpec((NC, N // 10, CP), lambda i: (0, i, 0)),
              pl.BlockSpec((1, CP), lambda i: (0, 0))],
    out_specs=pl.BlockSpec((N // 10, C), lambda i: (i, 0)),
    out_shape=jax.ShapeDtypeStruct((N, C), jnp.float32),
)


def kernel(x, edge_index, W1, b1, W2, b2):
    src2d = edge_index[0].reshape(E // CH, CH)
    dst2d = edge_index[1].reshape(E // CH, CH)
    z16 = jnp.zeros((RPT, H), jnp.float32)
    zcp = jnp.zeros((RPT, CP), jnp.float32)
    w2p = jnp.pad(W2, ((0, 0), (0, CP - C)))
    b2p = jnp.pad(b2, (0, CP - C)).reshape(1, CP)

    h1 = _mm1(x, W1)
    p1 = _seg16(src2d, dst2d, h1, z16)
    h2 = _mid(p1, b1.reshape(1, H), w2p)
    p2 = _segcp(src2d, dst2d, h2, zcp)
    return _post(p2, b2p)
